# Initial kernel scaffold; baseline (speedup 1.0000x reference)
import jax
import jax.numpy as jnp
from jax.experimental import pallas as pl


def kernel(positions, hidden_states, input_ln_scale, post_ln_scale, q_norm_scale, k_norm_scale, wq, wk, wv, wo, gate_kernel, w_gate, w_up, w_down):
    raise NotImplementedError("write your pallas kernel here")


# speedup vs baseline: 1.0481x; 1.0481x over previous
import functools

import jax
import jax.numpy as jnp
from jax import lax
from jax.experimental import pallas as pl
from jax.experimental.pallas import tpu as pltpu
from jax.experimental.pallas import tpu_sc as plsc

HID = 1024
N_HEADS = 16
N_KV = 4
HD = 64
E = 64
DFF = 512
EPS = 1e-06
BASE = 1000000.0
CAP = 256
SCALING = HD ** -0.5
T = 2048
TB = 256
RB = 128
TRASH = E * CAP


def _ln_body(x_ref, s_ref, o_ref):
    x = x_ref[...]
    var = jnp.mean(x * x, axis=-1, keepdims=True)
    o_ref[...] = x * lax.rsqrt(var + EPS) * s_ref[...]


def _qkv_body(h_ref, w_ref, pos_ref, sc_ref, o_ref, *, rot):
    h = h_ref[...]
    w = w_ref[0]
    q = jnp.dot(h, w, preferred_element_type=jnp.float32)
    if rot:
        var = jnp.mean(q * q, axis=-1, keepdims=True)
        q = q * lax.rsqrt(var + EPS) * sc_ref[...]
        lane = lax.broadcasted_iota(jnp.int32, (TB, HD), 1)
        half = HD // 2
        inv = 1.0 / (BASE ** ((lane % half).astype(jnp.float32) / half))
        f = pos_ref[...].astype(jnp.float32) * inv
        cos = jnp.cos(f)
        sin = jnp.where(lane < half, -jnp.sin(f), jnp.sin(f))
        q = q * cos + pltpu.roll(q, half, 1) * sin
    o_ref[0] = q

def _attn_body(q_ref, k_ref, v_ref, o_ref):
    qb = pl.program_id(1)
    q = q_ref[0]
    k = k_ref[0]
    s = lax.dot_general(q, k, (((1,), (1,)), ((), ())),
                        preferred_element_type=jnp.float32) * SCALING
    row = lax.broadcasted_iota(jnp.int32, (TB, T), 0) + qb * TB
    col = lax.broadcasted_iota(jnp.int32, (TB, T), 1)
    s = jnp.where(col > row, jnp.float32(-1e30), s)
    m = jnp.max(s, axis=-1, keepdims=True)
    e = jnp.exp(s - m)
    p = e / jnp.sum(e, axis=-1, keepdims=True)
    o_ref[0] = jnp.dot(p, v_ref[0], preferred_element_type=jnp.float32)


def _post_body(a_ref, wo_ref, hs_ref, ps_ref, gk_ref, x_ref, h2_ref, sc_ref):
    ao = jnp.dot(a_ref[...], wo_ref[...], preferred_element_type=jnp.float32)
    x = hs_ref[...] + ao
    var = jnp.mean(x * x, axis=-1, keepdims=True)
    h2 = x * lax.rsqrt(var + EPS) * ps_ref[...]
    logits = jnp.dot(h2, gk_ref[...], preferred_element_type=jnp.float32)
    x_ref[...] = x
    h2_ref[...] = h2
    sc_ref[...] = jax.nn.sigmoid(logits)


def _route_body(s_ref, d1_ref, d2_ref, s1_ref, s2_ref, w1_ref, w2_ref,
                cnt_ref, carry_ref):
    pid = pl.program_id(0)

    @pl.when(pid == 0)
    def _():
        carry_ref[...] = jnp.zeros((8, E), jnp.float32)

    s = s_ref[...]
    lane = lax.broadcasted_iota(jnp.int32, (RB, E), 1)
    m1 = jnp.max(s, axis=-1, keepdims=True)
    i1 = jnp.min(jnp.where(s == m1, lane, E), axis=-1, keepdims=True)
    oh1 = (lane == i1).astype(jnp.float32)
    s2 = jnp.where(lane == i1, jnp.float32(-1e30), s)
    m2 = jnp.max(s2, axis=-1, keepdims=True)
    i2 = jnp.min(jnp.where(s2 == m2, lane, E), axis=-1, keepdims=True)
    oh2 = (lane == i2).astype(jnp.float32)
    wsum = m1 + m2 + 1e-20
    w1 = m1 / wsum
    w2 = m2 / wsum

    S = oh1 + oh2
    r = lax.broadcasted_iota(jnp.int32, (RB, RB), 0)
    c = lax.broadcasted_iota(jnp.int32, (RB, RB), 1)
    tril = (c < r).astype(jnp.float32)
    cum = jnp.dot(tril, S, preferred_element_type=jnp.float32)
    cum = cum + carry_ref[0:1, :]
    p1 = jnp.sum(oh1 * cum, axis=-1, keepdims=True).astype(jnp.int32)
    p2 = jnp.sum(oh2 * (cum + oh1), axis=-1, keepdims=True).astype(jnp.int32)
    carry_ref[0:1, :] = carry_ref[0:1, :] + jnp.sum(S, axis=0, keepdims=True)
    cnt_ref[...] = carry_ref[0:1, :]

    k1 = p1 < CAP
    k2 = p2 < CAP
    d1_ref[...] = jnp.where(k1, i1 * CAP + p1, TRASH)
    d2_ref[...] = jnp.where(k2, i2 * CAP + p2, TRASH)
    s1_ref[...] = jnp.where(k1, i1 * CAP + p1, 0)
    s2_ref[...] = jnp.where(k2, i2 * CAP + p2, 0)
    w1_ref[...] = jnp.where(k1, w1, 0.0)
    w2_ref[...] = jnp.where(k2, w2, 0.0)

def _ffn_body(b_ref, wg_ref, wu_ref, wd_ref, cnt_ref, o_ref):
    e = pl.program_id(0)
    b = b_ref[...]
    a1 = jnp.dot(b, wg_ref[0], preferred_element_type=jnp.float32)
    a2 = jnp.dot(b, wu_ref[0], preferred_element_type=jnp.float32)
    h = a2 * (a1 * jax.nn.sigmoid(a1))
    y = jnp.dot(h, wd_ref[0], preferred_element_type=jnp.float32)
    lane = lax.broadcasted_iota(jnp.int32, (1, E), 1)
    cnt_e = jnp.sum(jnp.where(lane == e, cnt_ref[...], 0.0)).astype(jnp.int32)
    row = lax.broadcasted_iota(jnp.int32, (CAP, 1), 0)
    o_ref[...] = jnp.where(row < cnt_e, y, 0.0)


NW = 32
TW = T // NW
TC2 = TW // 2


def _dispatch_sc(h2_hbm, d1_hbm, d2_hbm, buf_hbm, i1v, i2v, xv, sem1, sem2):
    wid = lax.axis_index("s") * 2 + lax.axis_index("c")
    base = wid * TW
    pltpu.sync_copy(h2_hbm.at[pl.ds(base, TW)], xv)
    pltpu.sync_copy(d1_hbm.at[pl.ds(base, TW)], i1v)
    pltpu.sync_copy(d2_hbm.at[pl.ds(base, TW)], i2v)
    c1 = pltpu.async_copy(xv, buf_hbm.at[i1v], sem1)
    c2 = pltpu.async_copy(xv, buf_hbm.at[i2v], sem2)
    c1.wait()
    c2.wait()


def _combine_sc(x_hbm, yb_hbm, s1_hbm, s2_hbm, w1_hbm, w2_hbm, out_hbm,
                s1v, s2v, wv1, wv2, acc, g1, g2, sem1, sem2):
    wid = lax.axis_index("s") * 2 + lax.axis_index("c")
    gdn = jax.lax.GatherDimensionNumbers(
        offset_dims=(), collapsed_slice_dims=(0,), start_index_map=(0,))

    for chunk in range(2):
        b = wid * TW + chunk * TC2
        pltpu.sync_copy(x_hbm.at[pl.ds(b, TC2)], acc)
        pltpu.sync_copy(s1_hbm.at[pl.ds(b, TC2)], s1v)
        pltpu.sync_copy(s2_hbm.at[pl.ds(b, TC2)], s2v)
        pltpu.sync_copy(w1_hbm.at[pl.ds(b, TC2)], wv1)
        pltpu.sync_copy(w2_hbm.at[pl.ds(b, TC2)], wv2)
        pltpu.async_copy(yb_hbm.at[s1v], g1, sem1).wait()
        pltpu.async_copy(yb_hbm.at[s2v], g2, sem2).wait()

        for sub in range(TC2 // 16):
            w1c = wv1[pl.ds(sub * 16, 16)]
            w2c = wv2[pl.ds(sub * 16, 16)]

            def body(j, _, w1c=w1c, w2c=w2c, sub=sub):
                idx = jnp.full((16, 1), j, jnp.int32)
                w1j = lax.gather(w1c, idx, gdn, (1,),
                                 mode=lax.GatherScatterMode.PROMISE_IN_BOUNDS)
                w2j = lax.gather(w2c, idx, gdn, (1,),
                                 mode=lax.GatherScatterMode.PROMISE_IN_BOUNDS)
                r = sub * 16 + j
                for cc in range(HID // 16):
                    sl = pl.ds(cc * 16, 16)
                    acc[r, sl] = (acc[r, sl] + g1[r, sl] * w1j
                                  + g2[r, sl] * w2j)
                return 0

            lax.fori_loop(0, 16, body, 0)
        pltpu.sync_copy(acc, out_hbm.at[pl.ds(b, TC2)])

def _f32(shape):
    return jax.ShapeDtypeStruct(shape, jnp.float32)


def _i32(shape):
    return jax.ShapeDtypeStruct(shape, jnp.int32)


def kernel(positions, hidden_states, input_ln_scale, post_ln_scale,
           q_norm_scale, k_norm_scale, wq, wk, wv, wo, gate_kernel,
           w_gate, w_up, w_down):
    pos2 = positions.reshape(T, 1)
    wq3 = wq.reshape(HID, N_HEADS, HD).transpose(1, 0, 2)
    wk3 = wk.reshape(HID, N_KV, HD).transpose(1, 0, 2)
    wv3 = wv.reshape(HID, N_KV, HD).transpose(1, 0, 2)

    h = pl.pallas_call(
        _ln_body,
        grid=(T // TB,),
        in_specs=[pl.BlockSpec((TB, HID), lambda i: (i, 0)),
                  pl.BlockSpec((1, HID), lambda i: (0, 0))],
        out_specs=pl.BlockSpec((TB, HID), lambda i: (i, 0)),
        out_shape=_f32((T, HID)),
    )(hidden_states, input_ln_scale.reshape(1, HID))

    def qkv_call(w3, nh, scale, rot):
        return pl.pallas_call(
            functools.partial(_qkv_body, rot=rot),
            grid=(T // TB, nh),
            in_specs=[pl.BlockSpec((TB, HID), lambda i, hh: (i, 0)),
                      pl.BlockSpec((1, HID, HD), lambda i, hh: (hh, 0, 0)),
                      pl.BlockSpec((TB, 1), lambda i, hh: (i, 0)),
                      pl.BlockSpec((1, HD), lambda i, hh: (0, 0))],
            out_specs=pl.BlockSpec((1, TB, HD), lambda i, hh: (hh, i, 0)),
            out_shape=_f32((nh, T, HD)),
        )(h, w3, pos2, scale.reshape(1, HD))

    q3 = qkv_call(wq3, N_HEADS, q_norm_scale, True)
    k3 = qkv_call(wk3, N_KV, k_norm_scale, True)
    v3 = qkv_call(wv3, N_KV, k_norm_scale, False)

    attn3 = pl.pallas_call(
        _attn_body,
        grid=(N_HEADS, T // TB),
        in_specs=[pl.BlockSpec((1, TB, HD), lambda hh, i: (hh, i, 0)),
                  pl.BlockSpec((1, T, HD), lambda hh, i: (hh // 4, 0, 0)),
                  pl.BlockSpec((1, T, HD), lambda hh, i: (hh // 4, 0, 0))],
        out_specs=pl.BlockSpec((1, TB, HD), lambda hh, i: (hh, i, 0)),
        out_shape=_f32((N_HEADS, T, HD)),
    )(q3, k3, v3)

    attn = attn3.transpose(1, 0, 2).reshape(T, N_HEADS * HD)

    x, h2, sc = pl.pallas_call(
        _post_body,
        grid=(T // TB,),
        in_specs=[pl.BlockSpec((TB, HID), lambda i: (i, 0)),
                  pl.BlockSpec((HID, HID), lambda i: (0, 0)),
                  pl.BlockSpec((TB, HID), lambda i: (i, 0)),
                  pl.BlockSpec((1, HID), lambda i: (0, 0)),
                  pl.BlockSpec((HID, E), lambda i: (0, 0))],
        out_specs=[pl.BlockSpec((TB, HID), lambda i: (i, 0)),
                   pl.BlockSpec((TB, HID), lambda i: (i, 0)),
                   pl.BlockSpec((TB, E), lambda i: (i, 0))],
        out_shape=[_f32((T, HID)), _f32((T, HID)), _f32((T, E))],
    )(attn, wo, hidden_states, post_ln_scale.reshape(1, HID), gate_kernel)

    rspec = pl.BlockSpec((RB, 1), lambda i: (i, 0))
    d1, d2, s1, s2, w1, w2, cnt = pl.pallas_call(
        _route_body,
        grid=(T // RB,),
        in_specs=[pl.BlockSpec((RB, E), lambda i: (i, 0))],
        out_specs=[rspec, rspec, rspec, rspec, rspec, rspec,
                   pl.BlockSpec((1, E), lambda i: (0, 0))],
        out_shape=[_i32((T, 1)), _i32((T, 1)), _i32((T, 1)), _i32((T, 1)),
                   _f32((T, 1)), _f32((T, 1)), _f32((1, E))],
        scratch_shapes=[pltpu.VMEM((8, E), jnp.float32)],
    )(sc)

    mesh = plsc.VectorSubcoreMesh(core_axis_name="c", subcore_axis_name="s")
    buf = pl.kernel(
        _dispatch_sc,
        mesh=mesh,
        out_type=_f32((TRASH + 8, HID)),
        scratch_types=[pltpu.VMEM((TW,), jnp.int32),
                       pltpu.VMEM((TW,), jnp.int32),
                       pltpu.VMEM((TW, HID), jnp.float32),
                       pltpu.SemaphoreType.DMA,
                       pltpu.SemaphoreType.DMA],
    )(h2, d1.reshape(T), d2.reshape(T))

    yb = pl.pallas_call(
        _ffn_body,
        grid=(E,),
        in_specs=[pl.BlockSpec((CAP, HID), lambda e: (e, 0)),
                  pl.BlockSpec((1, HID, DFF), lambda e: (e, 0, 0)),
                  pl.BlockSpec((1, HID, DFF), lambda e: (e, 0, 0)),
                  pl.BlockSpec((1, DFF, HID), lambda e: (e, 0, 0)),
                  pl.BlockSpec((1, E), lambda e: (0, 0))],
        out_specs=pl.BlockSpec((CAP, HID), lambda e: (e, 0)),
        out_shape=_f32((E * CAP, HID)),
    )(buf, w_gate, w_up, w_down, cnt)

    out = pl.kernel(
        _combine_sc,
        mesh=mesh,
        out_type=_f32((T, HID)),
        scratch_types=[pltpu.VMEM((TC2,), jnp.int32),
                       pltpu.VMEM((TC2,), jnp.int32),
                       pltpu.VMEM((TC2,), jnp.float32),
                       pltpu.VMEM((TC2,), jnp.float32),
                       pltpu.VMEM((TC2, HID), jnp.float32),
                       pltpu.VMEM((TC2, HID), jnp.float32),
                       pltpu.VMEM((TC2, HID), jnp.float32),
                       pltpu.SemaphoreType.DMA,
                       pltpu.SemaphoreType.DMA],
    )(x, yb, s1.reshape(T), s2.reshape(T), w1.reshape(T), w2.reshape(T))
    return out
